# Initial kernel scaffold; baseline (speedup 1.0000x reference)
import jax
import jax.numpy as jnp
from jax.experimental import pallas as pl


def kernel(x_seq, edge_index_seq, target_influencer_idx, W1l, b1l, W1r, W2l, b2l, W2r, W_ih, W_hh, b_ih, b_hh, w_att, b_att, Wp1, bp1, Wp2, bp2):
    raise NotImplementedError("write your pallas kernel here")


# speedup vs baseline: 3.2994x; 3.2994x over previous
import functools

import jax
import jax.numpy as jnp
from jax import lax
from jax.experimental import pallas as pl
from jax.experimental.pallas import tpu as pltpu
from jax.experimental.pallas import tpu_sc as plsc

T, N, E = 4, 10000, 320000
D_IN, HID, OUT = 128, 64, 32
NW = 32
EW = E // NW
CH = 2000
NCH = EW // CH
GPC = CH // 16
NROWS = 10112
RPT = NROWS // 16
CW = 16
NB = 10
BN = N // NB


@functools.lru_cache(maxsize=None)
def _make_phase1():
  mesh = plsc.VectorSubcoreMesh(core_axis_name="c", subcore_axis_name="s")

  @functools.partial(
      pl.kernel,
      mesh=mesh,
      compiler_params=pltpu.CompilerParams(
          needs_layout_passes=False, use_tc_tiling_on_sc=False),
      out_type=jax.ShapeDtypeStruct((T, 2, NROWS, CW), jnp.int32),
      scratch_types=[
          pltpu.VMEM((CH,), jnp.int32),
          pltpu.VMEM((CH,), jnp.int32),
          pltpu.VMEM((16, CW), jnp.int32),
          pltpu.VMEM((16, CW), jnp.int32),
          pltpu.VMEM((16,), jnp.int32),
          pltpu.VMEM_SHARED((NROWS, CW), jnp.int32),
      ],
  )
  def _sc_phase1(edges, tgt, c_out, dst_v, src_v, ones_v, zc_v, tgt_v, c_sh):
    ci = lax.axis_index("c")
    si = lax.axis_index("s")
    wid = si * 2 + ci
    col = lax.iota(jnp.int32, 16)
    zrow = jnp.zeros((16,), jnp.int32)
    onerow = jnp.where(col == 0, 1, 0).astype(jnp.int32)

    def _fill(i, _):
      zc_v[i, :] = zrow
      ones_v[i, :] = onerow
      return 0

    lax.fori_loop(0, 16, _fill, 0)
    pltpu.sync_copy(tgt.at[:], tgt_v)
    tg = tgt_v[...]
    trash = jnp.full((16,), N, jnp.int32)
    for t in range(T):
      def _zslab(i, _):
        pltpu.sync_copy(zc_v, c_sh.at[pl.ds(si * RPT + i * 16, 16)])
        return 0

      lax.fori_loop(0, RPT // 16, _zslab, 0)
      pltpu.sync_copy(zc_v.at[pl.ds(0, 8)],
                      c_sh.at[pl.ds(si * RPT + (RPT // 16) * 16, 8)])
      plsc.subcore_barrier()
      for ch in range(NCH):
        off_d = t * 2 * E + E + wid * EW + ch * CH
        pltpu.sync_copy(edges.at[pl.ds(off_d, CH)], dst_v)
        off_s = t * 2 * E + wid * EW + ch * CH
        pltpu.sync_copy(edges.at[pl.ds(off_s, CH)], src_v)

        def _grp(g, carry):
          dv = dst_v[pl.ds(g * 16, 16)]
          m = dv == tg
          anyf = plsc.all_reduce_population_count(m)[0] > 0

          @pl.when(anyf)
          def _():
            sv = src_v[pl.ds(g * 16, 16)]
            idx = jnp.where(m, sv, trash)
            pltpu.sync_copy(ones_v, c_sh.at[idx], add=True)

          return carry

        lax.fori_loop(0, GPC, _grp, 0)
      plsc.subcore_barrier()
      pltpu.sync_copy(c_sh.at[pl.ds(si * RPT, RPT)],
                      c_out.at[t, ci, pl.ds(si * RPT, RPT)])
      plsc.subcore_barrier()

  return _sc_phase1


@functools.lru_cache(maxsize=None)
def _make_phase2():
  mesh = plsc.VectorSubcoreMesh(core_axis_name="c", subcore_axis_name="s")

  @functools.partial(
      pl.kernel,
      mesh=mesh,
      compiler_params=pltpu.CompilerParams(
          needs_layout_passes=False, use_tc_tiling_on_sc=False),
      out_type=(jax.ShapeDtypeStruct((T, 2, NROWS, D_IN), jnp.float32),
                jax.ShapeDtypeStruct((T, 2, NROWS, CW), jnp.float32)),
      scratch_types=[
          pltpu.VMEM((CH,), jnp.int32),
          pltpu.VMEM((CH,), jnp.int32),
          pltpu.VMEM((N,), jnp.int32),
          pltpu.VMEM((16, D_IN), jnp.float32),
          pltpu.VMEM((16, D_IN), jnp.float32),
          pltpu.VMEM((16, CW), jnp.float32),
          pltpu.VMEM((16, CW), jnp.float32),
          pltpu.VMEM_SHARED((NROWS, D_IN), jnp.float32),
          pltpu.VMEM_SHARED((NROWS, CW), jnp.float32),
          pltpu.SemaphoreType.DMA,
      ],
  )
  def _sc_phase2(edges, needs, xflat, a_out, d_out, dst_v, src_v, nd_v,
                 rows_v, za_v, onesf_v, zd_v, a_sh, d_sh, sem):
    ci = lax.axis_index("c")
    si = lax.axis_index("s")
    wid = si * 2 + ci
    col = lax.iota(jnp.int32, 16)
    zrowf = jnp.zeros((16,), jnp.float32)
    onerowf = jnp.where(col == 0, 1.0, 0.0).astype(jnp.float32)

    def _fillw(i, _):
      za_v[i // 8, pl.ds((i % 8) * 16, 16)] = zrowf
      return 0

    lax.fori_loop(0, 16 * (D_IN // 16), _fillw, 0)

    def _filln(i, _):
      zd_v[i, :] = zrowf
      onesf_v[i, :] = onerowf
      return 0

    lax.fori_loop(0, 16, _filln, 0)
    trash = jnp.full((16,), N, jnp.int32)
    for t in range(T):
      def _zslab(i, _):
        pltpu.sync_copy(za_v, a_sh.at[pl.ds(si * RPT + i * 16, 16)])
        pltpu.sync_copy(zd_v, d_sh.at[pl.ds(si * RPT + i * 16, 16)])
        return 0

      lax.fori_loop(0, RPT // 16, _zslab, 0)
      base8 = si * RPT + (RPT // 16) * 16
      pltpu.sync_copy(za_v.at[pl.ds(0, 8)], a_sh.at[pl.ds(base8, 8)])
      pltpu.sync_copy(zd_v.at[pl.ds(0, 8)], d_sh.at[pl.ds(base8, 8)])
      plsc.subcore_barrier()
      pltpu.sync_copy(needs.at[pl.ds(t * N, N)], nd_v)
      for ch in range(NCH):
        off_d = t * 2 * E + E + wid * EW + ch * CH
        pltpu.sync_copy(edges.at[pl.ds(off_d, CH)], dst_v)
        off_s = t * 2 * E + wid * EW + ch * CH
        pltpu.sync_copy(edges.at[pl.ds(off_s, CH)], src_v)

        def _grp(g, carry):
          dv = dst_v[pl.ds(g * 16, 16)]
          fl = plsc.load_gather(nd_v, [dv])
          m = fl > 0
          anyf = plsc.all_reduce_population_count(m)[0] > 0

          @pl.when(anyf)
          def _():
            sv = src_v[pl.ds(g * 16, 16)]
            gidx = jnp.where(m, sv + t * N, 0)
            didx = jnp.where(m, dv, trash)
            pltpu.async_copy(xflat.at[gidx], rows_v, sem).wait()
            pltpu.sync_copy(rows_v, a_sh.at[didx], add=True)
            pltpu.sync_copy(onesf_v, d_sh.at[didx], add=True)

          return carry

        lax.fori_loop(0, GPC, _grp, 0)
      plsc.subcore_barrier()
      pltpu.sync_copy(a_sh.at[pl.ds(si * RPT, RPT)],
                      a_out.at[t, ci, pl.ds(si * RPT, RPT)])
      pltpu.sync_copy(d_sh.at[pl.ds(si * RPT, RPT)],
                      d_out.at[t, ci, pl.ds(si * RPT, RPT)])
      plsc.subcore_barrier()

  return _sc_phase2


def _tc_body(a_ref, d_ref, x_ref, c_ref, ts_ref, w1l, w1r, b1, w2l, b2, w2r,
             wih, whh, bih, bhh, watt, batt, wp1, bp1r, wp2, bp2r,
             out_ref, s2s, hts, degs):
  t = pl.program_id(0)
  b = pl.program_id(1)
  f32 = jnp.float32
  hi = lax.Precision.HIGHEST

  def dot(a, bb):
    return jnp.dot(a, bb, preferred_element_type=f32, precision=hi)

  @pl.when(b == 0)
  def _():
    s2s[pl.ds(t, 1)] = jnp.zeros((1, HID), f32)
    hts[pl.ds(t, 1)] = jnp.zeros((1, HID), f32)
    degs[t] = 0.0

  a = a_ref[0]
  d = d_ref[0]
  s = a[0] + a[1]
  deg1 = d[0, :, 0:1] + d[1, :, 0:1]
  agg = s / jnp.maximum(deg1, 1.0)
  xb = x_ref[0]
  h1 = jnp.maximum(dot(agg, w1l[...]) + dot(xb, w1r[...]) + b1[...], 0.0)
  cb = c_ref[0, 0]
  tb = ts_ref[0, 0]
  s2s[pl.ds(t, 1)] += dot(cb, h1)
  hts[pl.ds(t, 1)] += dot(tb, h1)
  degs[t] += jnp.sum(cb)

  @pl.when((t == T - 1) & (b == NB - 1))
  def _():
    embs = []
    for tt in range(T):
      s2 = s2s[pl.ds(tt, 1)] / jnp.maximum(degs[tt], 1.0)
      embs.append(dot(s2, w2l[...]) + b2[...]
                  + dot(hts[pl.ds(tt, 1)], w2r[...]))
    h = jnp.zeros((1, HID), f32)
    outs = []
    for tt in range(T):
      gi = dot(embs[tt], wih[...]) + bih[...]
      gh = dot(h, whh[...]) + bhh[...]
      r = jax.nn.sigmoid(gi[:, :HID] + gh[:, :HID])
      z = jax.nn.sigmoid(gi[:, HID:2 * HID] + gh[:, HID:2 * HID])
      nn = jnp.tanh(gi[:, 2 * HID:] + r * gh[:, 2 * HID:])
      h = (1.0 - z) * nn + z * h
      outs.append(h)
    logits = [dot(outs[tt], watt[...]) + batt[...] for tt in range(T)]
    mx = logits[0]
    for tt in range(1, T):
      mx = jnp.maximum(mx, logits[tt])
    es = [jnp.exp(l - mx) for l in logits]
    ssum = es[0] + es[1] + es[2] + es[3]
    ctx = es[0] / ssum * outs[0]
    for tt in range(1, T):
      ctx = ctx + es[tt] / ssum * outs[tt]
    hh = jnp.maximum(dot(ctx, wp1[...]) + bp1r[...], 0.0)
    sc = dot(hh, wp2[...]) + bp2r[...]
    out_ref[:, :] = sc


def kernel(x_seq, edge_index_seq, target_influencer_idx, W1l, b1l, W1r,
           W2l, b2l, W2r, W_ih, W_hh, b_ih, b_hh, w_att, b_att,
           Wp1, bp1, Wp2, bp2):
  f32 = jnp.float32
  edges_flat = edge_index_seq.astype(jnp.int32).reshape(-1)
  tgt = jnp.full((16,), target_influencer_idx, jnp.int32)
  xflat = x_seq.reshape(T * N, D_IN)

  c_parts = _make_phase1()(edges_flat, tgt)
  c_sum = c_parts[:, 0, :N, 0] + c_parts[:, 1, :N, 0]
  tmask = jnp.arange(N, dtype=jnp.int32) == target_influencer_idx
  needs = ((c_sum > 0) | tmask[None, :]).astype(jnp.int32).reshape(-1)

  a_parts, d_parts = _make_phase2()(edges_flat, needs, xflat)

  c_f = c_sum.astype(f32).reshape(T, NB, 1, BN)
  tsel = tmask.astype(f32).reshape(1, NB, 1, BN)

  def full2(shape):
    return pl.BlockSpec(shape, lambda t, b: (0, 0))

  score = pl.pallas_call(
      _tc_body,
      grid=(T, NB),
      in_specs=[
          pl.BlockSpec((1, 2, BN, D_IN), lambda t, b: (t, 0, b, 0)),
          pl.BlockSpec((1, 2, BN, CW), lambda t, b: (t, 0, b, 0)),
          pl.BlockSpec((1, BN, D_IN), lambda t, b: (t, b, 0)),
          pl.BlockSpec((1, 1, 1, BN), lambda t, b: (t, b, 0, 0)),
          pl.BlockSpec((1, 1, 1, BN), lambda t, b: (0, b, 0, 0)),
          full2((D_IN, HID)), full2((D_IN, HID)), full2((1, HID)),
          full2((HID, OUT)), full2((1, OUT)), full2((HID, OUT)),
          full2((OUT, 3 * HID)), full2((HID, 3 * HID)),
          full2((1, 3 * HID)), full2((1, 3 * HID)),
          full2((HID, 1)), full2((1, 1)),
          full2((HID, HID // 2)), full2((1, HID // 2)),
          full2((HID // 2, 1)), full2((1, 1)),
      ],
      out_specs=pl.BlockSpec((1, 1), lambda t, b: (0, 0)),
      out_shape=jax.ShapeDtypeStruct((1, 1), f32),
      scratch_shapes=[
          pltpu.VMEM((8, HID), f32),
          pltpu.VMEM((8, HID), f32),
          pltpu.SMEM((8,), f32),
      ],
  )(a_parts, d_parts, x_seq, c_f, tsel,
    W1l.T, W1r.T, b1l[None, :], W2l.T, b2l[None, :], W2r.T,
    W_ih.T, W_hh.T, b_ih[None, :], b_hh[None, :],
    w_att.T, b_att[None, :], Wp1.T, bp1[None, :], Wp2.T, bp2[None, :])
  return score[0, 0]
